# baseline (speedup 1.0000x reference)
import functools

import jax
import jax.numpy as jnp
from jax import lax
from jax.experimental import pallas as pl
from jax.experimental.pallas import tpu as pltpu
from jax.experimental.pallas import tpu_sc as plsc

VOCAB_N = 100000
SEQ_N = 2048
DIM_N = 768
BATCH_N = 4

NUM_CORES = 2
NUM_SUBCORES = 16
NUM_WORKERS = NUM_CORES * NUM_SUBCORES
ROWS_N = BATCH_N * SEQ_N
ROWS_PER_WORKER = ROWS_N // NUM_WORKERS
CHUNK = 64
NUM_CHUNKS = ROWS_PER_WORKER // CHUNK
LANES = 16


def _sc_embed(idx_hbm, table_hbm, pos_hbm, out_sum_hbm, out_raw_hbm,
              idx_v, raw_v, sum_v, gsem):
    wid = lax.axis_index("s") * NUM_CORES + lax.axis_index("c")
    base = wid * ROWS_PER_WORKER
    pos_base = lax.rem(base, SEQ_N)

    for c in range(NUM_CHUNKS):
        off = c * CHUNK
        pltpu.sync_copy(idx_hbm.at[pl.ds(base + off, CHUNK)], idx_v)
        gcopy = pltpu.async_copy(table_hbm.at[idx_v], raw_v, gsem)
        pltpu.sync_copy(pos_hbm.at[pl.ds(pos_base + off, CHUNK)], sum_v)
        gcopy.wait()

        def add_row(i, _):
            for j in range(DIM_N // LANES):
                sl = pl.ds(j * LANES, LANES)
                sum_v[i, sl] = sum_v[i, sl] + raw_v[i, sl]
            return 0

        lax.fori_loop(0, CHUNK, add_row, 0)

        pltpu.sync_copy(raw_v, out_raw_hbm.at[pl.ds(base + off, CHUNK)])
        pltpu.sync_copy(sum_v, out_sum_hbm.at[pl.ds(base + off, CHUNK)])


_sc_call = pl.kernel(
    _sc_embed,
    out_type=(
        jax.ShapeDtypeStruct((ROWS_N, DIM_N), jnp.float32),
        jax.ShapeDtypeStruct((ROWS_N, DIM_N), jnp.float32),
    ),
    mesh=plsc.VectorSubcoreMesh(core_axis_name="c", subcore_axis_name="s"),
    scratch_types=[
        pltpu.VMEM((CHUNK,), jnp.int32),
        pltpu.VMEM((CHUNK, DIM_N), jnp.float32),
        pltpu.VMEM((CHUNK, DIM_N), jnp.float32),
        pltpu.SemaphoreType.DMA,
    ],
)


@jax.jit
def kernel(inputs, embed_table, pos_embs):
    idx = inputs.reshape(-1).astype(jnp.int32)
    out_sum, out_raw = _sc_call(idx, embed_table, pos_embs)
    full = (BATCH_N, SEQ_N, DIM_N)
    return out_sum.reshape(full), out_raw.reshape(full)


# speedup vs baseline: 1.1772x; 1.1772x over previous
import jax
import jax.numpy as jnp
from jax import lax
from jax.experimental import pallas as pl
from jax.experimental.pallas import tpu as pltpu
from jax.experimental.pallas import tpu_sc as plsc

VOCAB_N = 100000
SEQ_N = 2048
DIM_N = 768
BATCH_N = 4

NUM_CORES = 2
NUM_SUBCORES = 16
NUM_WORKERS = NUM_CORES * NUM_SUBCORES
ROWS_N = BATCH_N * SEQ_N
SEQ_PER_WORKER = SEQ_N // NUM_WORKERS
CHUNK = 32
SUBSPANS = SEQ_PER_WORKER // CHUNK
NUM_CHUNKS = SUBSPANS * BATCH_N
LANES = 16


def _sc_embed(idx_hbm, table_hbm, pos_hbm, out_sum_hbm, out_raw_hbm,
              idx0_v, idx1_v, raw0_v, raw1_v, sum0_v, sum1_v, pos_v,
              gsem0, gsem1, wsem0, wsem1):
    wid = lax.axis_index("s") * NUM_CORES + lax.axis_index("c")
    seq_base = wid * SEQ_PER_WORKER
    idx_v = (idx0_v, idx1_v)
    raw_v = (raw0_v, raw1_v)
    sum_v = (sum0_v, sum1_v)
    gsem = (gsem0, gsem1)
    wsem = (wsem0, wsem1)

    def chunk_offsets(i):
        ss, b = divmod(i, BATCH_N)
        pos_off = seq_base + ss * CHUNK
        return b * SEQ_N + pos_off, pos_off, ss

    pltpu.sync_copy(pos_hbm.at[pl.ds(seq_base, CHUNK)], pos_v)
    row0, _, _ = chunk_offsets(0)
    pltpu.sync_copy(idx_hbm.at[pl.ds(row0, CHUNK)], idx_v[0])
    gathers = [pltpu.async_copy(table_hbm.at[idx_v[0]], raw_v[0], gsem[0]),
               None]
    writes = [None, None]

    for i in range(NUM_CHUNKS):
        p = i % 2
        row_off, _, ss = chunk_offsets(i)
        gathers[p].wait()

        if i + 1 < NUM_CHUNKS:
            q = 1 - p
            if writes[q] is not None:
                for w in writes[q]:
                    w.wait()
                writes[q] = None
            nrow, _, _ = chunk_offsets(i + 1)
            pltpu.sync_copy(idx_hbm.at[pl.ds(nrow, CHUNK)], idx_v[q])
            gathers[q] = pltpu.async_copy(table_hbm.at[idx_v[q]], raw_v[q],
                                          gsem[q])

        if i % BATCH_N == 0 and ss > 0:
            pltpu.sync_copy(
                pos_hbm.at[pl.ds(seq_base + ss * CHUNK, CHUNK)], pos_v)

        rv, sv = raw_v[p], sum_v[p]

        def add_row(r, _, rv=rv, sv=sv):
            for j in range(DIM_N // LANES):
                sl = pl.ds(j * LANES, LANES)
                sv[r, sl] = pos_v[r, sl] + rv[r, sl]
            return 0

        lax.fori_loop(0, CHUNK, add_row, 0)

        writes[p] = (
            pltpu.async_copy(rv, out_raw_hbm.at[pl.ds(row_off, CHUNK)],
                             wsem[p]),
            pltpu.async_copy(sv, out_sum_hbm.at[pl.ds(row_off, CHUNK)],
                             wsem[p]),
        )

    for ws in writes:
        if ws is not None:
            for w in ws:
                w.wait()


_sc_call = pl.kernel(
    _sc_embed,
    out_type=(
        jax.ShapeDtypeStruct((ROWS_N, DIM_N), jnp.float32),
        jax.ShapeDtypeStruct((ROWS_N, DIM_N), jnp.float32),
    ),
    mesh=plsc.VectorSubcoreMesh(core_axis_name="c", subcore_axis_name="s"),
    scratch_types=[
        pltpu.VMEM((CHUNK,), jnp.int32),
        pltpu.VMEM((CHUNK,), jnp.int32),
        pltpu.VMEM((CHUNK, DIM_N), jnp.float32),
        pltpu.VMEM((CHUNK, DIM_N), jnp.float32),
        pltpu.VMEM((CHUNK, DIM_N), jnp.float32),
        pltpu.VMEM((CHUNK, DIM_N), jnp.float32),
        pltpu.VMEM((CHUNK, DIM_N), jnp.float32),
        pltpu.SemaphoreType.DMA,
        pltpu.SemaphoreType.DMA,
        pltpu.SemaphoreType.DMA,
        pltpu.SemaphoreType.DMA,
    ],
)


@jax.jit
def kernel(inputs, embed_table, pos_embs):
    idx = inputs.reshape(-1).astype(jnp.int32)
    out_sum, out_raw = _sc_call(idx, embed_table, pos_embs)
    full = (BATCH_N, SEQ_N, DIM_N)
    return out_sum.reshape(full), out_raw.reshape(full)
